# Initial kernel scaffold; baseline (speedup 1.0000x reference)
import jax
import jax.numpy as jnp
from jax.experimental import pallas as pl


def kernel(seq_pads, seq_lens, Wr, br, Ws, bs, Wq, bq):
    raise NotImplementedError("write your pallas kernel here")


# speedup vs baseline: 18.2414x; 18.2414x over previous
import math

import jax
import jax.numpy as jnp
from jax.experimental import pallas as pl
from jax.experimental.pallas import tpu as pltpu

B = 16
L = 512
H = 256
M = 3
T_END = 100.0
TAUS = (1.0, 10.0, 100.0)
LNT = tuple(math.log(t) for t in TAUS)
C = 32
NCHUNK = L // C
GRID = NCHUNK + 1


def _scan_body(sp_ref, lens_ref, ars_ref, aq_ref,
               bef_ref, aft_ref, delta_ref, hhat_ref, dch_ref):
    i = pl.program_id(0)

    @pl.when(i == 0)
    def _init():
        sp = sp_ref[...]
        lens = lens_ref[...]
        diffs = jnp.concatenate([sp[:, :1], sp[:, 1:] - sp[:, :-1]], axis=1)
        diffs_ext = jnp.concatenate(
            [diffs, jnp.zeros((B, 1), jnp.float32)], axis=1)
        pos = jax.lax.broadcasted_iota(jnp.int32, (B, L + 1), 1)
        posL = jax.lax.broadcasted_iota(jnp.int32, (B, L), 1)
        t_last = jnp.sum(jnp.where(posL == lens - 1, sp, 0.0),
                         axis=1, keepdims=True)
        delta = jnp.where(pos < lens, diffs_ext, -1.0)
        delta = jnp.where(pos == lens, T_END - t_last, delta)
        delta_ref[...] = delta
        padded = jnp.concatenate(
            [delta, jnp.zeros((B, GRID * C - (L + 1)), jnp.float32)], axis=1)
        for j in range(GRID):
            dch_ref[j] = padded[:, j * C:(j + 1) * C]
        hhat_ref[...] = jnp.zeros((M, B, H), jnp.float32)

    @pl.when(i < NCHUNK)
    def _chunk():
        dch = dch_ref[i]
        edt = [jnp.exp(dch * (-1.0 / TAUS[m])) for m in range(M)]
        h = [hhat_ref[m] for m in range(M)]
        ones = jnp.ones((B, 1), jnp.float32)
        a_rs = ars_ref[...]
        a_q = aq_ref[...]
        for k in range(C):
            dtk = dch[:, k:k + 1]
            dec = [h[m] * edt[m][:, k:k + 1] for m in range(M)]
            for m in range(M):
                aft_ref[m, k, :, :] = h[m]
                bef_ref[m, k, :, :] = dec[m]
            h_comb = dec[0] + dec[1] + dec[2]
            x1 = jnp.concatenate([h_comb, dtk, ones], axis=1)
            ln_rs = jax.lax.dot(x1, a_rs,
                                preferred_element_type=jnp.float32)
            ln_r = ln_rs[:, :H]
            ln_s = ln_rs[:, H:]
            a = [-(ln_r - LNT[m]) ** 2 for m in range(M)]
            amax = jnp.maximum(jnp.maximum(a[0], a[1]), a[2])
            r = [jnp.exp(a[m] - amax) for m in range(M)]
            rsum = r[0] + r[1] + r[2]
            h_ret = (r[0] * dec[0] + r[1] * dec[1] + r[2] * dec[2]) / rsum
            x2 = jnp.concatenate([h_ret, dtk, ones], axis=1)
            q = jnp.tanh(jax.lax.dot(x2, a_q,
                                     preferred_element_type=jnp.float32))
            c = [-(ln_s - LNT[m]) ** 2 for m in range(M)]
            cmax = jnp.maximum(jnp.maximum(c[0], c[1]), c[2])
            s = [jnp.exp(c[m] - cmax) for m in range(M)]
            ssum = s[0] + s[1] + s[2]
            h = [dec[m] + (s[m] / ssum) * (q - dec[m]) for m in range(M)]
        for m in range(M):
            hhat_ref[m] = h[m]

    @pl.when(i == NCHUNK)
    def _epilogue():
        dt_last = dch_ref[NCHUNK][:, 0:1]
        for m in range(M):
            hm = hhat_ref[m]
            aft_ref[m, 0, :, :] = hm
            bef_ref[m, 0, :, :] = hm * jnp.exp(dt_last * (-1.0 / TAUS[m]))


def kernel(seq_pads, seq_lens, Wr, br, Ws, bs, Wq, bq):
    sp = seq_pads.reshape(B, L)
    lens = seq_lens.astype(jnp.int32).reshape(B, 1)
    a_rs = jnp.concatenate(
        [jnp.concatenate([Wr[1:], Wr[:1], br[None, :]], axis=0),
         jnp.concatenate([Ws[1:], Ws[:1], bs[None, :]], axis=0)],
        axis=1)
    a_q = jnp.concatenate([Wq[1:], Wq[:1], bq[None, :]], axis=0)

    bef_t, aft_t, delta = pl.pallas_call(
        _scan_body,
        grid=(GRID,),
        in_specs=[
            pl.BlockSpec((B, L), lambda i: (0, 0)),
            pl.BlockSpec((B, 1), lambda i: (0, 0)),
            pl.BlockSpec((H + 2, 2 * H), lambda i: (0, 0)),
            pl.BlockSpec((H + 2, H), lambda i: (0, 0)),
        ],
        out_specs=[
            pl.BlockSpec((M, C, B, H), lambda i: (0, i, 0, 0)),
            pl.BlockSpec((M, C, B, H), lambda i: (0, i, 0, 0)),
            pl.BlockSpec((B, L + 1), lambda i: (0, 0)),
        ],
        out_shape=[
            jax.ShapeDtypeStruct((M, L + 1, B, H), jnp.float32),
            jax.ShapeDtypeStruct((M, L + 1, B, H), jnp.float32),
            jax.ShapeDtypeStruct((B, L + 1), jnp.float32),
        ],
        scratch_shapes=[
            pltpu.VMEM((M, B, H), jnp.float32),
            pltpu.VMEM((GRID, B, C), jnp.float32),
        ],
        compiler_params=pltpu.CompilerParams(
            dimension_semantics=("arbitrary",)),
    )(sp, lens, a_rs, a_q)

    befores = jnp.transpose(bef_t, (0, 2, 1, 3))
    afters = jnp.transpose(aft_t, (0, 2, 1, 3))
    return befores, afters, delta[:, :, None]


# speedup vs baseline: 19.7019x; 1.0801x over previous
import math

import jax
import jax.numpy as jnp
from jax.experimental import pallas as pl
from jax.experimental.pallas import tpu as pltpu

B = 16
L = 512
H = 256
M = 3
T_END = 100.0
TAUS = (1.0, 10.0, 100.0)
LNT = tuple(math.log(t) for t in TAUS)
C = 32
NCHUNK = L // C
GRID = NCHUNK + 1


def _scan_body(sp_ref, lens_ref, whrs_ref, whq_ref, xb_ref,
               bef_ref, aft_ref, delta_ref, hhat_ref, dch_ref):
    i = pl.program_id(0)

    @pl.when(i == 0)
    def _init():
        sp = sp_ref[...]
        lens = lens_ref[...]
        diffs = jnp.concatenate([sp[:, :1], sp[:, 1:] - sp[:, :-1]], axis=1)
        diffs_ext = jnp.concatenate(
            [diffs, jnp.zeros((B, 1), jnp.float32)], axis=1)
        pos = jax.lax.broadcasted_iota(jnp.int32, (B, L + 1), 1)
        posL = jax.lax.broadcasted_iota(jnp.int32, (B, L), 1)
        t_last = jnp.sum(jnp.where(posL == lens - 1, sp, 0.0),
                         axis=1, keepdims=True)
        delta = jnp.where(pos < lens, diffs_ext, -1.0)
        delta = jnp.where(pos == lens, T_END - t_last, delta)
        delta_ref[...] = delta
        padded = jnp.concatenate(
            [delta, jnp.zeros((B, GRID * C - (L + 1)), jnp.float32)], axis=1)
        for j in range(GRID):
            dch_ref[j] = padded[:, j * C:(j + 1) * C]
        hhat_ref[...] = jnp.zeros((M, B, H), jnp.float32)

    @pl.when(i < NCHUNK)
    def _chunk():
        dch = dch_ref[i]
        edt = [jnp.exp(dch * (-1.0 / TAUS[m])) for m in range(M)]
        h = [hhat_ref[m] for m in range(M)]
        wh_rs = whrs_ref[...]
        wh_q = whq_ref[...]
        wx_rs = xb_ref[0:1, :]
        b_rs = xb_ref[1:2, :]
        wx_q = xb_ref[2:3, :H]
        b_q = xb_ref[3:4, :H]
        for k in range(C):
            dtk = dch[:, k:k + 1]
            pre_rs = dtk * wx_rs + b_rs
            pre_q = dtk * wx_q + b_q
            dec = [h[m] * edt[m][:, k:k + 1] for m in range(M)]
            for m in range(M):
                aft_ref[m, k, :, :] = h[m]
                bef_ref[m, k, :, :] = dec[m]
            h_comb = dec[0] + dec[1] + dec[2]
            ln_rs = jax.lax.dot(h_comb, wh_rs,
                                preferred_element_type=jnp.float32) + pre_rs
            ln_r = ln_rs[:, :H]
            ln_s = ln_rs[:, H:]
            a = [-(ln_r - LNT[m]) ** 2 for m in range(M)]
            amax = jnp.maximum(jnp.maximum(a[0], a[1]), a[2])
            r = [jnp.exp(a[m] - amax) for m in range(M)]
            rsum = r[0] + r[1] + r[2]
            h_ret = (r[0] * dec[0] + r[1] * dec[1] + r[2] * dec[2]) / rsum
            q = jnp.tanh(jax.lax.dot(h_ret, wh_q,
                                     preferred_element_type=jnp.float32)
                         + pre_q)
            c = [-(ln_s - LNT[m]) ** 2 for m in range(M)]
            cmax = jnp.maximum(jnp.maximum(c[0], c[1]), c[2])
            s = [jnp.exp(c[m] - cmax) for m in range(M)]
            ssum = s[0] + s[1] + s[2]
            h = [dec[m] + (s[m] / ssum) * (q - dec[m]) for m in range(M)]
        for m in range(M):
            hhat_ref[m] = h[m]

    @pl.when(i == NCHUNK)
    def _epilogue():
        dt_last = dch_ref[NCHUNK][:, 0:1]
        for m in range(M):
            hm = hhat_ref[m]
            aft_ref[m, 0, :, :] = hm
            bef_ref[m, 0, :, :] = hm * jnp.exp(dt_last * (-1.0 / TAUS[m]))


def kernel(seq_pads, seq_lens, Wr, br, Ws, bs, Wq, bq):
    sp = seq_pads.reshape(B, L)
    lens = seq_lens.astype(jnp.int32).reshape(B, 1)
    wh_rs = jnp.concatenate([Wr[1:], Ws[1:]], axis=1)
    wh_q = Wq[1:]
    zpad = jnp.zeros((H,), jnp.float32)
    xb = jnp.stack([
        jnp.concatenate([Wr[0], Ws[0]]),
        jnp.concatenate([br, bs]),
        jnp.concatenate([Wq[0], zpad]),
        jnp.concatenate([bq, zpad]),
    ] + [jnp.zeros((2 * H,), jnp.float32)] * 4, axis=0)

    bef_t, aft_t, delta = pl.pallas_call(
        _scan_body,
        grid=(GRID,),
        in_specs=[
            pl.BlockSpec((B, L), lambda i: (0, 0)),
            pl.BlockSpec((B, 1), lambda i: (0, 0)),
            pl.BlockSpec((H, 2 * H), lambda i: (0, 0)),
            pl.BlockSpec((H, H), lambda i: (0, 0)),
            pl.BlockSpec((8, 2 * H), lambda i: (0, 0)),
        ],
        out_specs=[
            pl.BlockSpec((M, C, B, H), lambda i: (0, i, 0, 0)),
            pl.BlockSpec((M, C, B, H), lambda i: (0, i, 0, 0)),
            pl.BlockSpec((B, L + 1), lambda i: (0, 0)),
        ],
        out_shape=[
            jax.ShapeDtypeStruct((M, L + 1, B, H), jnp.float32),
            jax.ShapeDtypeStruct((M, L + 1, B, H), jnp.float32),
            jax.ShapeDtypeStruct((B, L + 1), jnp.float32),
        ],
        scratch_shapes=[
            pltpu.VMEM((M, B, H), jnp.float32),
            pltpu.VMEM((GRID, B, C), jnp.float32),
        ],
        compiler_params=pltpu.CompilerParams(
            dimension_semantics=("arbitrary",)),
    )(sp, lens, wh_rs, wh_q, xb)

    befores = jnp.transpose(bef_t, (0, 2, 1, 3))
    afters = jnp.transpose(aft_t, (0, 2, 1, 3))
    return befores, afters, delta[:, :, None]
